# Initial kernel scaffold; baseline (speedup 1.0000x reference)
import jax
import jax.numpy as jnp
from jax.experimental import pallas as pl


def kernel(x, edge_index, edge_attr, x_emb1, x_emb2, e_emb1, e_emb2, W1, b1, W2, b2, gamma, beta):
    raise NotImplementedError("write your pallas kernel here")


# speedup vs baseline: 2.9556x; 2.9556x over previous
import functools

import jax
import jax.numpy as jnp
from jax import lax

jax.config.update("jax_default_matmul_precision", "highest")
from jax.experimental import pallas as pl
from jax.experimental.pallas import tpu as pltpu
from jax.experimental.pallas import tpu_sc as plsc

N = 10000
E = 320000
D = 128
NC = 2
NS = 16
NW = NC * NS
CHUNK = 128
CPW = 80
EPW = CPW * CHUNK
E_PAD = NW * EPW
AGG_ROWS = 10240
RPT = AGG_ROWS // NS


def _sc_agg_body(h_hbm, src_hbm, dst_hbm, zero_hbm, out_hbm, idx_s, idx_d, rows, agg_sh):
    c = lax.axis_index("c")
    s = lax.axis_index("s")
    wid = s * NC + c
    pltpu.sync_copy(zero_hbm, agg_sh.at[pl.ds(s * RPT, RPT)])
    pltpu.sync_copy(src_hbm.at[pl.ds(wid * CPW, CPW)], idx_s)
    pltpu.sync_copy(dst_hbm.at[pl.ds(wid * CPW, CPW)], idx_d)
    plsc.subcore_barrier()

    def body(j, carry):
        pltpu.sync_copy(h_hbm.at[idx_s.at[j]], rows)
        pltpu.sync_copy(rows, agg_sh.at[idx_d.at[j]], add=True)
        return carry

    lax.fori_loop(0, CPW, body, 0)
    plsc.subcore_barrier()
    pltpu.sync_copy(agg_sh.at[pl.ds(s * RPT, RPT)],
                    out_hbm.at[c, pl.ds(s * RPT, RPT)])


def _sc_hist_body(dst_hbm, bins_hbm, eye_hbm, zero_hbm, out_hbm,
                  idx_d, bins_v, oh, cnt_sh):
    c = lax.axis_index("c")
    s = lax.axis_index("s")
    wid = s * NC + c
    pltpu.sync_copy(zero_hbm, cnt_sh.at[pl.ds(s * RPT, RPT)])
    pltpu.sync_copy(dst_hbm.at[pl.ds(wid * CPW, CPW)], idx_d)
    pltpu.sync_copy(bins_hbm.at[pl.ds(wid * CPW, CPW)], bins_v)
    plsc.subcore_barrier()

    def body(j, carry):
        pltpu.sync_copy(eye_hbm.at[bins_v.at[j]], oh)
        pltpu.sync_copy(oh, cnt_sh.at[idx_d.at[j]], add=True)
        return carry

    lax.fori_loop(0, CPW, body, 0)
    plsc.subcore_barrier()
    pltpu.sync_copy(cnt_sh.at[pl.ds(s * RPT, RPT)],
                    out_hbm.at[c, pl.ds(s * RPT, RPT)])


@functools.lru_cache(maxsize=None)
def _sc_kernels():
    mesh = plsc.VectorSubcoreMesh(core_axis_name="c", subcore_axis_name="s",
                                  num_cores=NC, num_subcores=NS)
    agg = pl.kernel(
        _sc_agg_body,
        out_type=jax.ShapeDtypeStruct((NC, AGG_ROWS, D), jnp.float32),
        mesh=mesh,
        scratch_types=[
            pltpu.VMEM((CPW, CHUNK), jnp.int32),
            pltpu.VMEM((CPW, CHUNK), jnp.int32),
            pltpu.VMEM((CHUNK, D), jnp.float32),
            pltpu.VMEM_SHARED((AGG_ROWS, D), jnp.float32),
        ],
    )
    hist = pl.kernel(
        _sc_hist_body,
        out_type=jax.ShapeDtypeStruct((NC, AGG_ROWS, D), jnp.float32),
        mesh=mesh,
        scratch_types=[
            pltpu.VMEM((CPW, CHUNK), jnp.int32),
            pltpu.VMEM((CPW, CHUNK), jnp.int32),
            pltpu.VMEM((CHUNK, D), jnp.float32),
            pltpu.VMEM_SHARED((AGG_ROWS, D), jnp.float32),
        ],
    )
    return agg, hist


def _tc_embed_body(x_ref, t1_ref, t2_ref, out_ref):
    x1 = x_ref[:, 0:1]
    x2 = x_ref[:, 1:2]
    oh1 = (x1 == lax.broadcasted_iota(jnp.int32, (N, 128), 1)).astype(jnp.float32)
    oh2 = (x2 == lax.broadcasted_iota(jnp.int32, (N, 8), 1)).astype(jnp.float32)
    h = jnp.dot(oh1, t1_ref[...], preferred_element_type=jnp.float32,
                precision=lax.Precision.HIGHEST)
    h = h + jnp.dot(oh2, t2_ref[...], preferred_element_type=jnp.float32,
                precision=lax.Precision.HIGHEST)
    out_ref[...] = h


_tc_embed = pl.pallas_call(
    _tc_embed_body,
    out_shape=jax.ShapeDtypeStruct((N, D), jnp.float32),
)


def _tc_mlp_body(p0, p1, h, c0, c1, m, sl, w1, bb1, w2, bb2, g, bt, out, *, last):
    agg = p0[...] + p1[...] + h[...] + sl[...]
    cnt = c0[...] + c1[...]
    agg = agg + jnp.dot(cnt, m[...], preferred_element_type=jnp.float32,
                precision=lax.Precision.HIGHEST)
    t = jnp.dot(agg, w1[...], preferred_element_type=jnp.float32,
                precision=lax.Precision.HIGHEST) + bb1[...]
    t = jnp.maximum(t, 0.0)
    t = jnp.dot(t, w2[...], preferred_element_type=jnp.float32,
                precision=lax.Precision.HIGHEST) + bb2[...]
    mean = jnp.mean(t, axis=0, keepdims=True)
    d = t - mean
    var = jnp.mean(d * d, axis=0, keepdims=True)
    r = d * lax.rsqrt(var + 1e-5) * g[...] + bt[...]
    if not last:
        r = jnp.maximum(r, 0.0)
    out[...] = r


_tc_mlp = {
    last: pl.pallas_call(
        functools.partial(_tc_mlp_body, last=last),
        out_shape=jax.ShapeDtypeStruct((N, D), jnp.float32),
    )
    for last in (False, True)
}


def kernel(x, edge_index, edge_attr, x_emb1, x_emb2, e_emb1, e_emb2,
           W1, b1, W2, b2, gamma, beta):
    pad = E_PAD - E
    src2 = jnp.concatenate(
        [edge_index[0], jnp.zeros((pad,), jnp.int32)]).reshape(NW * CPW, CHUNK)
    dst2 = jnp.concatenate(
        [edge_index[1], jnp.full((pad,), N, jnp.int32)]).reshape(NW * CPW, CHUNK)
    bins = edge_attr[:, 0] * 3 + edge_attr[:, 1]
    bins2 = jnp.concatenate(
        [bins, jnp.zeros((pad,), jnp.int32)]).reshape(NW * CPW, CHUNK)

    zeroD = jnp.zeros((RPT, D), jnp.float32)

    i9 = jnp.arange(9)
    M = e_emb1[:, :3, :][:, i9 // 3, :] + e_emb2[:, i9 % 3, :]
    M = jnp.pad(M, ((0, 0), (0, 7), (0, 0)))
    sconst = e_emb1[:, 4, :] + e_emb2[:, 0, :]

    t1 = jnp.pad(x_emb1, ((0, 8), (0, 0)))
    t2 = jnp.pad(x_emb2, ((0, 5), (0, 0)))

    _sc_agg, _sc_hist = _sc_kernels()
    h = _tc_embed(x, t1, t2)
    eye = jnp.pad(jnp.eye(16, dtype=jnp.float32), ((0, 0), (0, D - 16)))
    cnt = _sc_hist(dst2, bins2, eye, zeroD)
    c0 = cnt[0, :N, :16]
    c1 = cnt[1, :N, :16]

    L = W1.shape[0]
    for l in range(L):
        parts = _sc_agg(h, src2, dst2, zeroD)
        h = _tc_mlp[l == L - 1](
            parts[0, :N], parts[1, :N], h, c0, c1, M[l],
            sconst[l][None, :], W1[l], b1[l][None, :], W2[l],
            b2[l][None, :], gamma[l][None, :], beta[l][None, :])
    return h


# speedup vs baseline: 4.3109x; 1.4585x over previous
import functools

import jax
import jax.numpy as jnp
from jax import lax

jax.config.update("jax_default_matmul_precision", "highest")
from jax.experimental import pallas as pl
from jax.experimental.pallas import tpu as pltpu
from jax.experimental.pallas import tpu_sc as plsc

N = 10000
E = 320000
D = 128
NC = 2
NS = 16
NW = NC * NS
CHUNK = 128
CPW = 80
EPW = CPW * CHUNK
E_PAD = NW * EPW
AGG_ROWS = 10240
RPT = AGG_ROWS // NS


PHW = CPW // 2


def _gs_phase(table_hbm, gidx, sidx, rows0, rows1, sem0, sem1, acc_sh):
    def start(j, buf, sem):
        pltpu.async_copy(table_hbm.at[gidx.at[j]], buf, sem)

    def wait(j, buf, sem):
        pltpu.make_async_copy(table_hbm.at[gidx.at[j]], buf, sem).wait()

    def scat(j, buf):
        pltpu.sync_copy(buf, acc_sh.at[sidx.at[j]], add=True)

    start(0, rows0, sem0)

    def pair(i, carry):
        j = 2 * i
        start(j + 1, rows1, sem1)
        wait(j, rows0, sem0)
        scat(j, rows0)
        start(j + 2, rows0, sem0)
        wait(j + 1, rows1, sem1)
        scat(j + 1, rows1)
        return carry

    lax.fori_loop(0, PHW // 2 - 1, pair, 0)
    j = PHW - 2
    start(j + 1, rows1, sem1)
    wait(j, rows0, sem0)
    scat(j, rows0)
    wait(j + 1, rows1, sem1)
    scat(j + 1, rows1)


def _sc_agg_body(h_hbm, src_hbm, dst_hbm, zero_hbm, out_hbm,
                 idx_s, idx_d, rows0, rows1, agg_sh, sem0, sem1):
    c = lax.axis_index("c")
    s = lax.axis_index("s")
    wid = s * NC + c
    pltpu.sync_copy(zero_hbm, agg_sh.at[pl.ds(s * RPT, RPT)])
    plsc.subcore_barrier()
    for b in (0, 1):
        pltpu.sync_copy(src_hbm.at[pl.ds(wid * CPW + b * PHW, PHW)], idx_s)
        pltpu.sync_copy(dst_hbm.at[pl.ds(wid * CPW + b * PHW, PHW)], idx_d)
        _gs_phase(h_hbm, idx_s, idx_d, rows0, rows1, sem0, sem1, agg_sh)
    plsc.subcore_barrier()
    pltpu.sync_copy(agg_sh.at[pl.ds(s * RPT, RPT)],
                    out_hbm.at[c, pl.ds(s * RPT, RPT)])


def _sc_hist_body(dst_hbm, bins_hbm, eye_hbm, zero_hbm, out_hbm,
                  idx_d, bins_v, rows0, rows1, cnt_sh, sem0, sem1):
    c = lax.axis_index("c")
    s = lax.axis_index("s")
    wid = s * NC + c
    pltpu.sync_copy(zero_hbm, cnt_sh.at[pl.ds(s * RPT, RPT)])
    plsc.subcore_barrier()
    for b in (0, 1):
        pltpu.sync_copy(dst_hbm.at[pl.ds(wid * CPW + b * PHW, PHW)], idx_d)
        pltpu.sync_copy(bins_hbm.at[pl.ds(wid * CPW + b * PHW, PHW)], bins_v)
        _gs_phase(eye_hbm, bins_v, idx_d, rows0, rows1, sem0, sem1, cnt_sh)
    plsc.subcore_barrier()
    pltpu.sync_copy(cnt_sh.at[pl.ds(s * RPT, RPT)],
                    out_hbm.at[c, pl.ds(s * RPT, RPT)])


@functools.lru_cache(maxsize=None)
def _sc_kernels():
    mesh = plsc.VectorSubcoreMesh(core_axis_name="c", subcore_axis_name="s",
                                  num_cores=NC, num_subcores=NS)
    agg = pl.kernel(
        _sc_agg_body,
        out_type=jax.ShapeDtypeStruct((NC, AGG_ROWS, D), jnp.float32),
        mesh=mesh,
        scratch_types=[
            pltpu.VMEM((CPW // 2, CHUNK), jnp.int32),
            pltpu.VMEM((CPW // 2, CHUNK), jnp.int32),
            pltpu.VMEM((CHUNK, D), jnp.float32),
            pltpu.VMEM((CHUNK, D), jnp.float32),
            pltpu.VMEM_SHARED((AGG_ROWS, D), jnp.float32),
            pltpu.SemaphoreType.DMA,
            pltpu.SemaphoreType.DMA,
        ],
    )
    hist = pl.kernel(
        _sc_hist_body,
        out_type=jax.ShapeDtypeStruct((NC, AGG_ROWS, D), jnp.float32),
        mesh=mesh,
        scratch_types=[
            pltpu.VMEM((CPW // 2, CHUNK), jnp.int32),
            pltpu.VMEM((CPW // 2, CHUNK), jnp.int32),
            pltpu.VMEM((CHUNK, D), jnp.float32),
            pltpu.VMEM((CHUNK, D), jnp.float32),
            pltpu.VMEM_SHARED((AGG_ROWS, D), jnp.float32),
            pltpu.SemaphoreType.DMA,
            pltpu.SemaphoreType.DMA,
        ],
    )
    return agg, hist


def _tc_embed_body(x_ref, t1_ref, t2_ref, out_ref):
    x1 = x_ref[:, 0:1]
    x2 = x_ref[:, 1:2]
    oh1 = (x1 == lax.broadcasted_iota(jnp.int32, (N, 128), 1)).astype(jnp.float32)
    oh2 = (x2 == lax.broadcasted_iota(jnp.int32, (N, 8), 1)).astype(jnp.float32)
    h = jnp.dot(oh1, t1_ref[...], preferred_element_type=jnp.float32,
                precision=lax.Precision.HIGHEST)
    h = h + jnp.dot(oh2, t2_ref[...], preferred_element_type=jnp.float32,
                precision=lax.Precision.HIGHEST)
    out_ref[...] = h


_tc_embed = pl.pallas_call(
    _tc_embed_body,
    out_shape=jax.ShapeDtypeStruct((N, D), jnp.float32),
)


def _tc_mlp_body(p0, p1, h, c0, c1, m, sl, w1, bb1, w2, bb2, g, bt, out, *, last):
    agg = p0[...] + p1[...] + h[...] + sl[...]
    cnt = c0[...] + c1[...]
    agg = agg + jnp.dot(cnt, m[...], preferred_element_type=jnp.float32,
                precision=lax.Precision.HIGHEST)
    t = jnp.dot(agg, w1[...], preferred_element_type=jnp.float32,
                precision=lax.Precision.HIGHEST) + bb1[...]
    t = jnp.maximum(t, 0.0)
    t = jnp.dot(t, w2[...], preferred_element_type=jnp.float32,
                precision=lax.Precision.HIGHEST) + bb2[...]
    mean = jnp.mean(t, axis=0, keepdims=True)
    d = t - mean
    var = jnp.mean(d * d, axis=0, keepdims=True)
    r = d * lax.rsqrt(var + 1e-5) * g[...] + bt[...]
    if not last:
        r = jnp.maximum(r, 0.0)
    out[...] = r


_tc_mlp = {
    last: pl.pallas_call(
        functools.partial(_tc_mlp_body, last=last),
        out_shape=jax.ShapeDtypeStruct((N, D), jnp.float32),
    )
    for last in (False, True)
}


def kernel(x, edge_index, edge_attr, x_emb1, x_emb2, e_emb1, e_emb2,
           W1, b1, W2, b2, gamma, beta):
    pad = E_PAD - E
    src2 = jnp.concatenate(
        [edge_index[0], jnp.zeros((pad,), jnp.int32)]).reshape(NW * CPW, CHUNK)
    dst2 = jnp.concatenate(
        [edge_index[1], jnp.full((pad,), N, jnp.int32)]).reshape(NW * CPW, CHUNK)
    bins = (edge_attr[:, 0] * 3 + edge_attr[:, 1]) * 16 + (
        jnp.arange(E, dtype=jnp.int32) % 16)
    bins2 = jnp.concatenate(
        [bins, jnp.zeros((pad,), jnp.int32)]).reshape(NW * CPW, CHUNK)

    zeroD = jnp.zeros((RPT, D), jnp.float32)

    i9 = jnp.arange(9)
    M = e_emb1[:, :3, :][:, i9 // 3, :] + e_emb2[:, i9 % 3, :]
    M = jnp.pad(M, ((0, 0), (0, 7), (0, 0)))
    sconst = e_emb1[:, 4, :] + e_emb2[:, 0, :]

    t1 = jnp.pad(x_emb1, ((0, 8), (0, 0)))
    t2 = jnp.pad(x_emb2, ((0, 5), (0, 0)))

    _sc_agg, _sc_hist = _sc_kernels()
    h = _tc_embed(x, t1, t2)
    eye = jnp.repeat(
        jnp.pad(jnp.eye(16, dtype=jnp.float32), ((0, 0), (0, D - 16))),
        16, axis=0)
    cnt = _sc_hist(dst2, bins2, eye, zeroD)
    c0 = cnt[0, :N, :16]
    c1 = cnt[1, :N, :16]

    L = W1.shape[0]
    for l in range(L):
        parts = _sc_agg(h, src2, dst2, zeroD)
        h = _tc_mlp[l == L - 1](
            parts[0, :N], parts[1, :N], h, c0, c1, M[l],
            sconst[l][None, :], W1[l], b1[l][None, :], W2[l],
            b2[l][None, :], gamma[l][None, :], beta[l][None, :])
    return h
